# baseline (speedup 1.0000x reference)
import functools

import jax
import jax.numpy as jnp
from jax import lax
from jax.experimental import pallas as pl
from jax.experimental.pallas import tpu as pltpu
from jax.experimental.pallas import tpu_sc as plsc

N_TOKENS = 4096
N_IN = 1024
N_OUT = 1024
N_EXPERTS = 8
TILE_M = 256
P_MAX = N_TOKENS * 2 + N_EXPERTS * TILE_M
NT = P_MAX // TILE_M
NW = 32
TOK_PER_W = N_TOKENS // NW
DSUB = 32
NDSUB = TOK_PER_W // DSUB
CSUB = 16
NCSUB = TOK_PER_W // CSUB


def _cumsum_rows(x):
    n = x.shape[0]
    sh = 1
    while sh < n:
        pad = jnp.zeros((sh, x.shape[1]), x.dtype)
        x = x + jnp.concatenate([pad, x[:-sh, :]], axis=0)
        sh *= 2
    return x


def _routing_body(x_ref, wsel_ref, bsel_ref,
                  xb_ref, pos0_ref, pos1_ref, g0_ref, g1_ref, te_ref,
                  aux_ref):
    xb = x_ref[...].astype(jnp.bfloat16)
    xb_ref[...] = xb
    wselb = wsel_ref[...].astype(jnp.bfloat16)
    logits = (
        jnp.dot(xb, wselb, preferred_element_type=jnp.float32) + bsel_ref[...]
    )
    m = jnp.max(logits, axis=-1, keepdims=True)
    e = jnp.exp(logits - m)
    s = jnp.sum(e, axis=-1, keepdims=True)
    p = e / s

    row_sums = jnp.sum(p, axis=-1)
    mean = jnp.mean(row_sums)
    var = jnp.mean((row_sums - mean) ** 2)
    aux_ref[...] = (var / (mean * mean + 1e-10)).reshape(1, 1)

    iota = lax.broadcasted_iota(jnp.int32, p.shape, 1)
    max1 = jnp.max(p, axis=-1, keepdims=True)
    i1 = jnp.min(jnp.where(p == max1, iota, N_EXPERTS), axis=-1, keepdims=True)
    m1 = iota == i1
    p2 = jnp.where(m1, -1.0, p)
    max2 = jnp.max(p2, axis=-1, keepdims=True)
    i2 = jnp.min(jnp.where(p2 == max2, iota, N_EXPERTS), axis=-1, keepdims=True)
    m2 = iota == i2

    g0_ref[...] = jnp.broadcast_to(max1, (N_TOKENS, 16))
    g1_ref[...] = jnp.broadcast_to(max2, (N_TOKENS, 16))

    mask = (m1 | m2).astype(jnp.int32)
    incl = _cumsum_rows(mask)
    cnt = incl[N_TOKENS - 1:N_TOKENS, :]
    cntp = ((cnt + (TILE_M - 1)) // TILE_M) * TILE_M
    sh = 1
    acc = cntp
    while sh < N_EXPERTS:
        pad = jnp.zeros((1, sh), jnp.int32)
        acc = acc + jnp.concatenate([pad, acc[:, :-sh]], axis=1)
        sh *= 2
    base = acc - cntp

    rank = incl - mask
    pos_e = base + rank
    pos0_ref[...] = jnp.sum(m1.astype(jnp.int32) * pos_e, axis=-1)
    pos1_ref[...] = jnp.sum(m2.astype(jnp.int32) * pos_e, axis=-1)

    ends = base + cntp
    j = lax.broadcasted_iota(jnp.int32, (64, 1), 0) * TILE_M
    te = jnp.sum((j >= ends).astype(jnp.int32), axis=-1)
    te_ref[...] = te


def _grouped_body(te_ref, xg_ref, w_ref, b_ref, yg_ref):
    i = pl.program_id(0)

    @pl.when(te_ref[i] < N_EXPERTS)
    def _():
        wb = w_ref[0].astype(jnp.bfloat16)
        y = jnp.dot(xg_ref[...], wb, preferred_element_type=jnp.float32)
        yg_ref[...] = jnp.maximum(y + b_ref[0], 0.0)


def _make_dispatch():
    mesh = plsc.VectorSubcoreMesh(core_axis_name="c", subcore_axis_name="s")

    @functools.partial(
        pl.kernel,
        mesh=mesh,
        out_type=jax.ShapeDtypeStruct((P_MAX, N_IN // 2), jnp.int32),
        scratch_types=[
            pltpu.VMEM((DSUB,), jnp.int32),
            pltpu.VMEM((DSUB,), jnp.int32),
            pltpu.VMEM((DSUB, N_IN // 2), jnp.int32),
            pltpu.SemaphoreType.DMA,
        ],
    )
    def dispatch(xb_hbm, pos0_hbm, pos1_hbm, xg_hbm, idx0_v, idx1_v, rows_v,
                 sem):
        wid = lax.axis_index("s") * 2 + lax.axis_index("c")
        for c in range(NDSUB):
            tok0 = wid * TOK_PER_W + c * DSUB
            pltpu.sync_copy(pos0_hbm.at[pl.ds(tok0, DSUB)], idx0_v)
            pltpu.sync_copy(pos1_hbm.at[pl.ds(tok0, DSUB)], idx1_v)
            pltpu.sync_copy(xb_hbm.at[pl.ds(tok0, DSUB)], rows_v)
            cp0 = pltpu.async_copy(rows_v, xg_hbm.at[idx0_v], sem)
            cp1 = pltpu.async_copy(rows_v, xg_hbm.at[idx1_v], sem)
            cp0.wait()
            cp1.wait()

    return dispatch


def _make_combine():
    mesh = plsc.VectorSubcoreMesh(core_axis_name="c", subcore_axis_name="s")

    @functools.partial(
        pl.kernel,
        mesh=mesh,
        out_type=jax.ShapeDtypeStruct((N_TOKENS, N_OUT), jnp.float32),
        scratch_types=[
            pltpu.VMEM((CSUB,), jnp.int32),
            pltpu.VMEM((CSUB,), jnp.int32),
            pltpu.VMEM((CSUB,), jnp.int32),
            pltpu.VMEM((CSUB,), jnp.int32),
            pltpu.VMEM((CSUB, N_OUT), jnp.float32),
            pltpu.VMEM((CSUB, N_OUT), jnp.float32),
            pltpu.VMEM((CSUB, N_OUT), jnp.float32),
            pltpu.VMEM((CSUB, N_OUT), jnp.float32),
            pltpu.VMEM((CSUB, 16), jnp.float32),
            pltpu.VMEM((CSUB, 16), jnp.float32),
            pltpu.VMEM((CSUB, 16), jnp.float32),
            pltpu.VMEM((CSUB, 16), jnp.float32),
            pltpu.VMEM((CSUB, N_OUT), jnp.float32),
            pltpu.SemaphoreType.DMA,
            pltpu.SemaphoreType.DMA,
        ],
    )
    def combine(yg_hbm, pos0_hbm, pos1_hbm, g0_hbm, g1_hbm, out_hbm,
                i0a, i1a, i0b, i1b, r0a, r1a, r0b, r1b,
                g0a, g1a, g0b, g1b, out_v, sema, semb):
        wid = lax.axis_index("s") * 2 + lax.axis_index("c")
        bufs = [
            (i0a, i1a, r0a, r1a, g0a, g1a, sema),
            (i0b, i1b, r0b, r1b, g0b, g1b, semb),
        ]

        def start(c):
            i0, i1, r0, r1, gv0, gv1, sem = bufs[c % 2]
            tok0 = wid * TOK_PER_W + c * CSUB
            pltpu.sync_copy(pos0_hbm.at[pl.ds(tok0, CSUB)], i0)
            pltpu.sync_copy(pos1_hbm.at[pl.ds(tok0, CSUB)], i1)
            pltpu.sync_copy(g0_hbm.at[pl.ds(tok0, CSUB)], gv0)
            pltpu.sync_copy(g1_hbm.at[pl.ds(tok0, CSUB)], gv1)
            cp0 = pltpu.async_copy(yg_hbm.at[i0], r0, sem)
            cp1 = pltpu.async_copy(yg_hbm.at[i1], r1, sem)
            return cp0, cp1

        cps = start(0)
        for c in range(NCSUB):
            nxt = start(c + 1) if c + 1 < NCSUB else None
            cps[0].wait()
            cps[1].wait()
            _, _, r0, r1, gv0, gv1, _ = bufs[c % 2]

            def token_body(t, _):
                gl = gv0[t]
                gr = gv1[t]
                for j in range(N_OUT // 16):
                    sl = pl.ds(j * 16, 16)
                    out_v[t, sl] = gl * r0[t, sl] + gr * r1[t, sl]
                return 0

            lax.fori_loop(0, CSUB, token_body, 0)
            tok0 = wid * TOK_PER_W + c * CSUB
            pltpu.sync_copy(out_v, out_hbm.at[pl.ds(tok0, CSUB)])
            cps = nxt

    return combine


@functools.partial(jax.jit, static_argnames=())
def kernel(inputs, Wsel, bsel, W, b):
    xb, pos0, pos1, g0, g1, te, aux = pl.pallas_call(
        _routing_body,
        out_shape=(
            jax.ShapeDtypeStruct((N_TOKENS, N_IN), jnp.bfloat16),
            jax.ShapeDtypeStruct((N_TOKENS,), jnp.int32),
            jax.ShapeDtypeStruct((N_TOKENS,), jnp.int32),
            jax.ShapeDtypeStruct((N_TOKENS, 16), jnp.float32),
            jax.ShapeDtypeStruct((N_TOKENS, 16), jnp.float32),
            jax.ShapeDtypeStruct((64,), jnp.int32),
            jax.ShapeDtypeStruct((1, 1), jnp.float32),
        ),
    )(inputs, Wsel, bsel.reshape(1, N_EXPERTS))

    xb32 = lax.bitcast_convert_type(
        xb.reshape(N_TOKENS, N_IN // 2, 2), jnp.int32)
    xg32 = _make_dispatch()(xb32, pos0, pos1)
    xg = lax.bitcast_convert_type(xg32, jnp.bfloat16).reshape(P_MAX, N_IN)

    yg = pl.pallas_call(
        _grouped_body,
        grid_spec=pltpu.PrefetchScalarGridSpec(
            num_scalar_prefetch=1,
            grid=(NT,),
            in_specs=[
                pl.BlockSpec((TILE_M, N_IN), lambda i, te: (i, 0)),
                pl.BlockSpec(
                    (1, N_IN, N_OUT),
                    lambda i, te: (jnp.minimum(te[i], N_EXPERTS - 1), 0, 0)),
                pl.BlockSpec(
                    (1, 1, N_OUT),
                    lambda i, te: (jnp.minimum(te[i], N_EXPERTS - 1), 0, 0)),
            ],
            out_specs=pl.BlockSpec((TILE_M, N_OUT), lambda i, te: (i, 0)),
        ),
        out_shape=jax.ShapeDtypeStruct((P_MAX, N_OUT), jnp.float32),
        compiler_params=pltpu.CompilerParams(
            dimension_semantics=("arbitrary",),
        ),
    )(te, xg, W, b.reshape(N_EXPERTS, 1, N_OUT))

    out = _make_combine()(yg, pos0, pos1, g0, g1)
    return (out, aux.reshape(()))


# speedup vs baseline: 2.7540x; 2.7540x over previous
import functools

import jax
import jax.numpy as jnp
from jax import lax
from jax.experimental import pallas as pl
from jax.experimental.pallas import tpu as pltpu
from jax.experimental.pallas import tpu_sc as plsc

N_TOKENS = 4096
N_IN = 1024
N_OUT = 1024
N_EXPERTS = 8
TILE_M = 256
P_MAX = N_TOKENS * 2 + N_EXPERTS * TILE_M
NT = P_MAX // TILE_M
NW = 32
TOK_PER_W = N_TOKENS // NW
DSUB = 32
NDSUB = TOK_PER_W // DSUB
CSUB = 16
NCSUB = TOK_PER_W // CSUB


def _cumsum_rows(x):
    n = x.shape[0]
    sh = 1
    while sh < n:
        pad = jnp.zeros((sh, x.shape[1]), x.dtype)
        x = x + jnp.concatenate([pad, x[:-sh, :]], axis=0)
        sh *= 2
    return x


def _routing_body(x_ref, wsel_ref, bsel_ref,
                  pos0_ref, pos1_ref, g0_ref, g1_ref, te_ref, aux_ref):
    xb = x_ref[...].astype(jnp.bfloat16)
    wselb = wsel_ref[...].astype(jnp.bfloat16)
    logits = (
        jnp.dot(xb, wselb, preferred_element_type=jnp.float32) + bsel_ref[...]
    )
    m = jnp.max(logits, axis=-1, keepdims=True)
    e = jnp.exp(logits - m)
    s = jnp.sum(e, axis=-1, keepdims=True)
    p = e / s

    row_sums = jnp.sum(p, axis=-1)
    mean = jnp.mean(row_sums)
    var = jnp.mean((row_sums - mean) ** 2)
    aux_ref[...] = (var / (mean * mean + 1e-10)).reshape(1, 1)

    iota = lax.broadcasted_iota(jnp.int32, p.shape, 1)
    max1 = jnp.max(p, axis=-1, keepdims=True)
    i1 = jnp.min(jnp.where(p == max1, iota, N_EXPERTS), axis=-1, keepdims=True)
    m1 = iota == i1
    p2 = jnp.where(m1, -1.0, p)
    max2 = jnp.max(p2, axis=-1, keepdims=True)
    i2 = jnp.min(jnp.where(p2 == max2, iota, N_EXPERTS), axis=-1, keepdims=True)
    m2 = iota == i2

    g0_ref[...] = jnp.broadcast_to(max1, (N_TOKENS, 16))
    g1_ref[...] = jnp.broadcast_to(max2, (N_TOKENS, 16))

    mask = (m1 | m2).astype(jnp.int32)
    incl = _cumsum_rows(mask)
    cnt = incl[N_TOKENS - 1:N_TOKENS, :]
    cntp = ((cnt + (TILE_M - 1)) // TILE_M) * TILE_M
    sh = 1
    acc = cntp
    while sh < N_EXPERTS:
        pad = jnp.zeros((1, sh), jnp.int32)
        acc = acc + jnp.concatenate([pad, acc[:, :-sh]], axis=1)
        sh *= 2
    base = acc - cntp

    rank = incl - mask
    pos_e = base + rank
    pos0_ref[...] = jnp.sum(m1.astype(jnp.int32) * pos_e, axis=-1)
    pos1_ref[...] = jnp.sum(m2.astype(jnp.int32) * pos_e, axis=-1)

    ends = base + cntp
    j = lax.broadcasted_iota(jnp.int32, (64, 1), 0) * TILE_M
    te = jnp.sum((j >= ends).astype(jnp.int32), axis=-1)
    te_ref[...] = te


def _grouped_body(te_ref, xg_ref, w_ref, b_ref, yg_ref):
    i = pl.program_id(0)

    @pl.when(te_ref[i] < N_EXPERTS)
    def _():
        wb = w_ref[0].astype(jnp.bfloat16)
        xgb = xg_ref[...].astype(jnp.bfloat16)
        y = jnp.dot(xgb, wb, preferred_element_type=jnp.float32)
        yg_ref[...] = jnp.maximum(y + b_ref[0], 0.0)


def _make_dispatch():
    mesh = plsc.VectorSubcoreMesh(core_axis_name="c", subcore_axis_name="s")

    @functools.partial(
        pl.kernel,
        mesh=mesh,
        out_type=jax.ShapeDtypeStruct((P_MAX, N_IN), jnp.float32),
        scratch_types=[
            pltpu.VMEM((DSUB,), jnp.int32),
            pltpu.VMEM((DSUB,), jnp.int32),
            pltpu.VMEM((DSUB, N_IN), jnp.float32),
            pltpu.SemaphoreType.DMA,
        ],
    )
    def dispatch(xb_hbm, pos0_hbm, pos1_hbm, xg_hbm, idx0_v, idx1_v, rows_v,
                 sem):
        wid = lax.axis_index("s") * 2 + lax.axis_index("c")
        for c in range(NDSUB):
            tok0 = wid * TOK_PER_W + c * DSUB
            pltpu.sync_copy(pos0_hbm.at[pl.ds(tok0, DSUB)], idx0_v)
            pltpu.sync_copy(pos1_hbm.at[pl.ds(tok0, DSUB)], idx1_v)
            pltpu.sync_copy(xb_hbm.at[pl.ds(tok0, DSUB)], rows_v)
            cp0 = pltpu.async_copy(rows_v, xg_hbm.at[idx0_v], sem)
            cp1 = pltpu.async_copy(rows_v, xg_hbm.at[idx1_v], sem)
            cp0.wait()
            cp1.wait()

    return dispatch


def _make_combine():
    mesh = plsc.VectorSubcoreMesh(core_axis_name="c", subcore_axis_name="s")

    @functools.partial(
        pl.kernel,
        mesh=mesh,
        out_type=jax.ShapeDtypeStruct((N_TOKENS, N_OUT), jnp.float32),
        scratch_types=[
            pltpu.VMEM((CSUB,), jnp.int32),
            pltpu.VMEM((CSUB,), jnp.int32),
            pltpu.VMEM((CSUB,), jnp.int32),
            pltpu.VMEM((CSUB,), jnp.int32),
            pltpu.VMEM((CSUB, N_OUT), jnp.float32),
            pltpu.VMEM((CSUB, N_OUT), jnp.float32),
            pltpu.VMEM((CSUB, N_OUT), jnp.float32),
            pltpu.VMEM((CSUB, N_OUT), jnp.float32),
            pltpu.VMEM((CSUB, 16), jnp.float32),
            pltpu.VMEM((CSUB, 16), jnp.float32),
            pltpu.VMEM((CSUB, 16), jnp.float32),
            pltpu.VMEM((CSUB, 16), jnp.float32),
            pltpu.VMEM((CSUB, N_OUT), jnp.float32),
            pltpu.SemaphoreType.DMA,
            pltpu.SemaphoreType.DMA,
        ],
    )
    def combine(yg_hbm, pos0_hbm, pos1_hbm, g0_hbm, g1_hbm, out_hbm,
                i0a, i1a, i0b, i1b, r0a, r1a, r0b, r1b,
                g0a, g1a, g0b, g1b, out_v, sema, semb):
        wid = lax.axis_index("s") * 2 + lax.axis_index("c")
        bufs = [
            (i0a, i1a, r0a, r1a, g0a, g1a, sema),
            (i0b, i1b, r0b, r1b, g0b, g1b, semb),
        ]

        def start(c):
            i0, i1, r0, r1, gv0, gv1, sem = bufs[c % 2]
            tok0 = wid * TOK_PER_W + c * CSUB
            pltpu.sync_copy(pos0_hbm.at[pl.ds(tok0, CSUB)], i0)
            pltpu.sync_copy(pos1_hbm.at[pl.ds(tok0, CSUB)], i1)
            pltpu.sync_copy(g0_hbm.at[pl.ds(tok0, CSUB)], gv0)
            pltpu.sync_copy(g1_hbm.at[pl.ds(tok0, CSUB)], gv1)
            cp0 = pltpu.async_copy(yg_hbm.at[i0], r0, sem)
            cp1 = pltpu.async_copy(yg_hbm.at[i1], r1, sem)
            return cp0, cp1

        cps = start(0)
        for c in range(NCSUB):
            nxt = start(c + 1) if c + 1 < NCSUB else None
            cps[0].wait()
            cps[1].wait()
            _, _, r0, r1, gv0, gv1, _ = bufs[c % 2]

            def token_body(t, _):
                gl = gv0[t]
                gr = gv1[t]
                for j in range(N_OUT // 16):
                    sl = pl.ds(j * 16, 16)
                    out_v[t, sl] = gl * r0[t, sl] + gr * r1[t, sl]
                return 0

            lax.fori_loop(0, CSUB, token_body, 0)
            tok0 = wid * TOK_PER_W + c * CSUB
            pltpu.sync_copy(out_v, out_hbm.at[pl.ds(tok0, CSUB)])
            cps = nxt

    return combine


@functools.partial(jax.jit, static_argnames=())
def kernel(inputs, Wsel, bsel, W, b):
    pos0, pos1, g0, g1, te, aux = pl.pallas_call(
        _routing_body,
        out_shape=(
            jax.ShapeDtypeStruct((N_TOKENS,), jnp.int32),
            jax.ShapeDtypeStruct((N_TOKENS,), jnp.int32),
            jax.ShapeDtypeStruct((N_TOKENS, 16), jnp.float32),
            jax.ShapeDtypeStruct((N_TOKENS, 16), jnp.float32),
            jax.ShapeDtypeStruct((64,), jnp.int32),
            jax.ShapeDtypeStruct((1, 1), jnp.float32),
        ),
    )(inputs, Wsel, bsel.reshape(1, N_EXPERTS))

    xg = _make_dispatch()(inputs, pos0, pos1)

    yg = pl.pallas_call(
        _grouped_body,
        grid_spec=pltpu.PrefetchScalarGridSpec(
            num_scalar_prefetch=1,
            grid=(NT,),
            in_specs=[
                pl.BlockSpec((TILE_M, N_IN), lambda i, te: (i, 0)),
                pl.BlockSpec(
                    (1, N_IN, N_OUT),
                    lambda i, te: (jnp.minimum(te[i], N_EXPERTS - 1), 0, 0)),
                pl.BlockSpec(
                    (1, 1, N_OUT),
                    lambda i, te: (jnp.minimum(te[i], N_EXPERTS - 1), 0, 0)),
            ],
            out_specs=pl.BlockSpec((TILE_M, N_OUT), lambda i, te: (i, 0)),
        ),
        out_shape=jax.ShapeDtypeStruct((P_MAX, N_OUT), jnp.float32),
        compiler_params=pltpu.CompilerParams(
            dimension_semantics=("arbitrary",),
        ),
    )(te, xg, W, b.reshape(N_EXPERTS, 1, N_OUT))

    out = _make_combine()(yg, pos0, pos1, g0, g1)
    return (out, aux.reshape(()))


# speedup vs baseline: 2.7684x; 1.0052x over previous
import functools

import jax
import jax.numpy as jnp
from jax import lax
from jax.experimental import pallas as pl
from jax.experimental.pallas import tpu as pltpu
from jax.experimental.pallas import tpu_sc as plsc

N_TOKENS = 4096
N_IN = 1024
N_OUT = 1024
N_EXPERTS = 8
TILE_M = 256
P_MAX = N_TOKENS * 2 + N_EXPERTS * TILE_M
NT = P_MAX // TILE_M
NW = 32
TOK_PER_W = N_TOKENS // NW
DSUB = 32
NDSUB = TOK_PER_W // DSUB
CSUB = 16
NCSUB = TOK_PER_W // CSUB


def _cumsum_rows(x):
    n = x.shape[0]
    sh = 1
    while sh < n:
        pad = jnp.zeros((sh, x.shape[1]), x.dtype)
        x = x + jnp.concatenate([pad, x[:-sh, :]], axis=0)
        sh *= 2
    return x


def _routing_body(x_ref, wsel_ref, bsel_ref,
                  pos0_ref, pos1_ref, g0_ref, g1_ref, te_ref, aux_ref):
    xb = x_ref[...].astype(jnp.bfloat16)
    wselb = wsel_ref[...].astype(jnp.bfloat16)
    logits = (
        jnp.dot(xb, wselb, preferred_element_type=jnp.float32) + bsel_ref[...]
    )
    m = jnp.max(logits, axis=-1, keepdims=True)
    e = jnp.exp(logits - m)
    s = jnp.sum(e, axis=-1, keepdims=True)
    p = e / s

    row_sums = jnp.sum(p, axis=-1)
    mean = jnp.mean(row_sums)
    var = jnp.mean((row_sums - mean) ** 2)
    aux_ref[...] = (var / (mean * mean + 1e-10)).reshape(1, 1)

    iota = lax.broadcasted_iota(jnp.int32, p.shape, 1)
    max1 = jnp.max(p, axis=-1, keepdims=True)
    i1 = jnp.min(jnp.where(p == max1, iota, N_EXPERTS), axis=-1, keepdims=True)
    m1 = iota == i1
    p2 = jnp.where(m1, -1.0, p)
    max2 = jnp.max(p2, axis=-1, keepdims=True)
    i2 = jnp.min(jnp.where(p2 == max2, iota, N_EXPERTS), axis=-1, keepdims=True)
    m2 = iota == i2

    g0_ref[...] = jnp.broadcast_to(max1, (N_TOKENS, 16))
    g1_ref[...] = jnp.broadcast_to(max2, (N_TOKENS, 16))

    mask = (m1 | m2).astype(jnp.int32)
    incl = _cumsum_rows(mask)
    cnt = incl[N_TOKENS - 1:N_TOKENS, :]
    cntp = ((cnt + (TILE_M - 1)) // TILE_M) * TILE_M
    sh = 1
    acc = cntp
    while sh < N_EXPERTS:
        pad = jnp.zeros((1, sh), jnp.int32)
        acc = acc + jnp.concatenate([pad, acc[:, :-sh]], axis=1)
        sh *= 2
    base = acc - cntp

    rank = incl - mask
    pos_e = base + rank
    pos0_ref[...] = jnp.sum(m1.astype(jnp.int32) * pos_e, axis=-1)
    pos1_ref[...] = jnp.sum(m2.astype(jnp.int32) * pos_e, axis=-1)

    ends = base + cntp
    j = lax.broadcasted_iota(jnp.int32, (64, 1), 0) * TILE_M
    te = jnp.sum((j >= ends).astype(jnp.int32), axis=-1)
    te_ref[...] = te


def _grouped_body(te_ref, xg_ref, w_ref, b_ref, yg_ref):
    i = pl.program_id(0)

    @pl.when(te_ref[i] < N_EXPERTS)
    def _():
        wb = w_ref[0].astype(jnp.bfloat16)
        xgb = xg_ref[...].astype(jnp.bfloat16)
        y = jnp.dot(xgb, wb, preferred_element_type=jnp.float32)
        yg_ref[...] = jnp.maximum(y + b_ref[0], 0.0)


def _make_dispatch():
    mesh = plsc.VectorSubcoreMesh(core_axis_name="c", subcore_axis_name="s")

    @functools.partial(
        pl.kernel,
        mesh=mesh,
        out_type=jax.ShapeDtypeStruct((P_MAX, N_IN), jnp.float32),
        scratch_types=[
            pltpu.VMEM((DSUB,), jnp.int32),
            pltpu.VMEM((DSUB,), jnp.int32),
            pltpu.VMEM((DSUB,), jnp.int32),
            pltpu.VMEM((DSUB,), jnp.int32),
            pltpu.VMEM((DSUB, N_IN), jnp.float32),
            pltpu.VMEM((DSUB, N_IN), jnp.float32),
            pltpu.SemaphoreType.DMA,
            pltpu.SemaphoreType.DMA,
            pltpu.SemaphoreType.DMA,
            pltpu.SemaphoreType.DMA,
        ],
    )
    def dispatch(x_hbm, pos0_hbm, pos1_hbm, xg_hbm,
                 i0a, i1a, i0b, i1b, rowsa, rowsb,
                 semla, semlb, semsa, semsb):
        wid = lax.axis_index("s") * 2 + lax.axis_index("c")
        bufs = [(i0a, i1a, rowsa, semla, semsa),
                (i0b, i1b, rowsb, semlb, semsb)]

        def load(c):
            i0, i1, rows, seml, _ = bufs[c % 2]
            tok0 = wid * TOK_PER_W + c * DSUB
            pltpu.sync_copy(pos0_hbm.at[pl.ds(tok0, DSUB)], i0)
            pltpu.sync_copy(pos1_hbm.at[pl.ds(tok0, DSUB)], i1)
            return pltpu.async_copy(x_hbm.at[pl.ds(tok0, DSUB)], rows, seml)

        cpl = [None, None]
        cps = [None, None]
        cpl[0] = load(0)
        for c in range(NDSUB):
            b = c % 2
            cpl[b].wait()
            i0, i1, rows, _, sems = bufs[b]
            sc0 = pltpu.async_copy(rows, xg_hbm.at[i0], sems)
            sc1 = pltpu.async_copy(rows, xg_hbm.at[i1], sems)
            if c + 1 < NDSUB:
                nb = (c + 1) % 2
                if cps[nb] is not None:
                    cps[nb][0].wait()
                    cps[nb][1].wait()
                cpl[nb] = load(c + 1)
            cps[b] = (sc0, sc1)
        for b in range(2):
            if cps[b] is not None:
                cps[b][0].wait()
                cps[b][1].wait()

    return dispatch


def _make_combine():
    mesh = plsc.VectorSubcoreMesh(core_axis_name="c", subcore_axis_name="s")

    @functools.partial(
        pl.kernel,
        mesh=mesh,
        out_type=jax.ShapeDtypeStruct((N_TOKENS, N_OUT), jnp.float32),
        scratch_types=[
            pltpu.VMEM((CSUB,), jnp.int32),
            pltpu.VMEM((CSUB,), jnp.int32),
            pltpu.VMEM((CSUB,), jnp.int32),
            pltpu.VMEM((CSUB,), jnp.int32),
            pltpu.VMEM((CSUB, N_OUT), jnp.float32),
            pltpu.VMEM((CSUB, N_OUT), jnp.float32),
            pltpu.VMEM((CSUB, N_OUT), jnp.float32),
            pltpu.VMEM((CSUB, N_OUT), jnp.float32),
            pltpu.VMEM((CSUB, 16), jnp.float32),
            pltpu.VMEM((CSUB, 16), jnp.float32),
            pltpu.VMEM((CSUB, 16), jnp.float32),
            pltpu.VMEM((CSUB, 16), jnp.float32),
            pltpu.VMEM((CSUB, N_OUT), jnp.float32),
            pltpu.SemaphoreType.DMA,
            pltpu.SemaphoreType.DMA,
        ],
    )
    def combine(yg_hbm, pos0_hbm, pos1_hbm, g0_hbm, g1_hbm, out_hbm,
                i0a, i1a, i0b, i1b, r0a, r1a, r0b, r1b,
                g0a, g1a, g0b, g1b, out_v, sema, semb):
        wid = lax.axis_index("s") * 2 + lax.axis_index("c")
        bufs = [
            (i0a, i1a, r0a, r1a, g0a, g1a, sema),
            (i0b, i1b, r0b, r1b, g0b, g1b, semb),
        ]

        def start(c):
            i0, i1, r0, r1, gv0, gv1, sem = bufs[c % 2]
            tok0 = wid * TOK_PER_W + c * CSUB
            pltpu.sync_copy(pos0_hbm.at[pl.ds(tok0, CSUB)], i0)
            pltpu.sync_copy(pos1_hbm.at[pl.ds(tok0, CSUB)], i1)
            pltpu.sync_copy(g0_hbm.at[pl.ds(tok0, CSUB)], gv0)
            pltpu.sync_copy(g1_hbm.at[pl.ds(tok0, CSUB)], gv1)
            cp0 = pltpu.async_copy(yg_hbm.at[i0], r0, sem)
            cp1 = pltpu.async_copy(yg_hbm.at[i1], r1, sem)
            return cp0, cp1

        cps = start(0)
        for c in range(NCSUB):
            nxt = start(c + 1) if c + 1 < NCSUB else None
            cps[0].wait()
            cps[1].wait()
            _, _, r0, r1, gv0, gv1, _ = bufs[c % 2]

            def token_body(t, _):
                gl = gv0[t]
                gr = gv1[t]
                for j in range(N_OUT // 16):
                    sl = pl.ds(j * 16, 16)
                    out_v[t, sl] = gl * r0[t, sl] + gr * r1[t, sl]
                return 0

            lax.fori_loop(0, CSUB, token_body, 0)
            tok0 = wid * TOK_PER_W + c * CSUB
            pltpu.sync_copy(out_v, out_hbm.at[pl.ds(tok0, CSUB)])
            cps = nxt

    return combine


@functools.partial(jax.jit, static_argnames=())
def kernel(inputs, Wsel, bsel, W, b):
    pos0, pos1, g0, g1, te, aux = pl.pallas_call(
        _routing_body,
        out_shape=(
            jax.ShapeDtypeStruct((N_TOKENS,), jnp.int32),
            jax.ShapeDtypeStruct((N_TOKENS,), jnp.int32),
            jax.ShapeDtypeStruct((N_TOKENS, 16), jnp.float32),
            jax.ShapeDtypeStruct((N_TOKENS, 16), jnp.float32),
            jax.ShapeDtypeStruct((64,), jnp.int32),
            jax.ShapeDtypeStruct((1, 1), jnp.float32),
        ),
    )(inputs, Wsel, bsel.reshape(1, N_EXPERTS))

    xg = _make_dispatch()(inputs, pos0, pos1)

    yg = pl.pallas_call(
        _grouped_body,
        grid_spec=pltpu.PrefetchScalarGridSpec(
            num_scalar_prefetch=1,
            grid=(NT,),
            in_specs=[
                pl.BlockSpec((TILE_M, N_IN), lambda i, te: (i, 0)),
                pl.BlockSpec(
                    (1, N_IN, N_OUT),
                    lambda i, te: (jnp.minimum(te[i], N_EXPERTS - 1), 0, 0)),
                pl.BlockSpec(
                    (1, 1, N_OUT),
                    lambda i, te: (jnp.minimum(te[i], N_EXPERTS - 1), 0, 0)),
            ],
            out_specs=pl.BlockSpec((TILE_M, N_OUT), lambda i, te: (i, 0)),
        ),
        out_shape=jax.ShapeDtypeStruct((P_MAX, N_OUT), jnp.float32),
        compiler_params=pltpu.CompilerParams(
            dimension_semantics=("arbitrary",),
        ),
    )(te, xg, W, b.reshape(N_EXPERTS, 1, N_OUT))

    out = _make_combine()(yg, pos0, pos1, g0, g1)
    return (out, aux.reshape(()))


# speedup vs baseline: 4.1108x; 1.4849x over previous
import functools

import jax
import jax.numpy as jnp
from jax.experimental import pallas as pl
from jax.experimental.pallas import tpu as pltpu

N_TOKENS = 4096
N_IN = 1024
N_OUT = 1024
N_EXPERTS = 8
ROW_CHUNK = 1024


def _moe_body(x_ref, wsel_ref, bsel_ref, w_ref, b_ref, out_ref, aux_ref,
              g_ref):
    i = pl.program_id(0)

    @pl.when(i == 0)
    def _routing():
        xb = x_ref[...].astype(jnp.bfloat16)
        wselb = wsel_ref[...].astype(jnp.bfloat16)
        logits = (
            jnp.dot(xb, wselb, preferred_element_type=jnp.float32)
            + bsel_ref[...]
        )
        m = jnp.max(logits, axis=-1, keepdims=True)
        e = jnp.exp(logits - m)
        s = jnp.sum(e, axis=-1, keepdims=True)
        p = e / s

        row_sums = jnp.sum(p, axis=-1)
        mean = jnp.mean(row_sums)
        var = jnp.mean((row_sums - mean) ** 2)
        aux_ref[...] = (var / (mean * mean + 1e-10)).reshape(1, 1)

        iota = jax.lax.broadcasted_iota(jnp.int32, p.shape, 1)
        max1 = jnp.max(p, axis=-1, keepdims=True)
        i1 = jnp.min(jnp.where(p == max1, iota, N_EXPERTS), axis=-1,
                     keepdims=True)
        m1 = iota == i1
        p2 = jnp.where(m1, -1.0, p)
        max2 = jnp.max(p2, axis=-1, keepdims=True)
        i2 = jnp.min(jnp.where(p2 == max2, iota, N_EXPERTS), axis=-1,
                     keepdims=True)
        m2 = iota == i2
        g_ref[...] = (max1 * m1.astype(jnp.float32)
                      + max2 * m2.astype(jnp.float32))

    wb = w_ref[0].astype(jnp.bfloat16)
    iota = jax.lax.broadcasted_iota(jnp.int32, (N_TOKENS, N_EXPERTS), 1)
    g = jnp.sum(
        g_ref[...] * (iota == i).astype(jnp.float32), axis=-1, keepdims=True
    )
    for c in range(N_TOKENS // ROW_CHUNK):
        rows = pl.ds(c * ROW_CHUNK, ROW_CHUNK)
        xbc = x_ref[rows, :].astype(jnp.bfloat16)
        y = jnp.dot(xbc, wb, preferred_element_type=jnp.float32) + b_ref[0]
        gy = jnp.maximum(y, 0.0) * g[c * ROW_CHUNK:(c + 1) * ROW_CHUNK, :]

        @pl.when(i == 0)
        def _():
            out_ref[rows, :] = gy

        @pl.when(i != 0)
        def _():
            out_ref[rows, :] += gy


@functools.partial(jax.jit, static_argnames=())
def kernel(inputs, Wsel, bsel, W, b):
    out, aux = pl.pallas_call(
        _moe_body,
        grid=(N_EXPERTS,),
        in_specs=[
            pl.BlockSpec((N_TOKENS, N_IN), lambda i: (0, 0)),
            pl.BlockSpec((N_IN, N_EXPERTS), lambda i: (0, 0)),
            pl.BlockSpec((1, N_EXPERTS), lambda i: (0, 0)),
            pl.BlockSpec((1, N_IN, N_OUT), lambda i: (i, 0, 0)),
            pl.BlockSpec((1, 1, N_OUT), lambda i: (i, 0, 0)),
        ],
        out_specs=(
            pl.BlockSpec((N_TOKENS, N_OUT), lambda i: (0, 0)),
            pl.BlockSpec((1, 1), lambda i: (0, 0)),
        ),
        out_shape=(
            jax.ShapeDtypeStruct((N_TOKENS, N_OUT), jnp.float32),
            jax.ShapeDtypeStruct((1, 1), jnp.float32),
        ),
        scratch_shapes=[pltpu.VMEM((N_TOKENS, N_EXPERTS), jnp.float32)],
        compiler_params=pltpu.CompilerParams(
            dimension_semantics=("arbitrary",),
        ),
    )(inputs, Wsel, bsel.reshape(1, N_EXPERTS), W,
      b.reshape(N_EXPERTS, 1, N_OUT))
    return (out, aux.reshape(()))
